# Initial kernel scaffold; baseline (speedup 1.0000x reference)
import jax
import jax.numpy as jnp
from jax.experimental import pallas as pl


def kernel(xyz, flow, params):
    raise NotImplementedError("write your pallas kernel here")


# speedup vs baseline: 10.6527x; 10.6527x over previous
import functools

import jax
import jax.numpy as jnp
from jax import lax
from jax.experimental import pallas as pl
from jax.experimental.pallas import tpu as pltpu
from jax.experimental.pallas import tpu_sc as plsc

EPS = 1e-5
K = 16
WN = 16
OC = 64
NW = 32
CHUNK = 128

def _dot(a, b):
    return jax.lax.dot_general(
        a.astype(jnp.bfloat16), b.astype(jnp.bfloat16),
        (((1,), (0,)), ((), ())), preferred_element_type=jnp.float32)


def _knn_body(qT_ref, rN_ref, o_ref, *, nn):
    b = pl.program_id(0)
    mq = qT_ref.shape[2]
    inf = jnp.float32(float("inf"))
    qs = None
    rs = None
    qr = None
    for d in range(3):
        qd = qT_ref[0, d:d + 1, :]
        rd = rN_ref[0, :, d:d + 1]
        qdb = qd.astype(jnp.bfloat16).astype(jnp.float32)
        rdb = rd.astype(jnp.bfloat16).astype(jnp.float32)
        qs = qd * qd if qs is None else qs + qd * qd
        rs = rd * rd if rs is None else rs + rd * rd
        qr = rdb * qdb if qr is None else qr + rdb * qdb
    d2 = (qs - 2.0 * qr) + rs
    sub_iota = lax.broadcasted_iota(jnp.int32, (nn, mq), 0)
    off = b * nn
    for j in range(K):
        mval = jnp.min(d2, axis=0, keepdims=True)
        idxj = jnp.min(jnp.where(d2 == mval, sub_iota, nn),
                       axis=0, keepdims=True)
        o_ref[0, j:j + 1, :] = idxj + off
        if j < K - 1:
            d2 = jnp.where(sub_iota == idxj, inf, d2)


def _knn(qT, rN, mq):
    b, _, m = qT.shape
    nn = rN.shape[1]
    out = pl.pallas_call(
        functools.partial(_knn_body, nn=nn),
        grid=(b, m // mq),
        in_specs=[
            pl.BlockSpec((1, 3, mq), lambda i, j: (i, 0, j)),
            pl.BlockSpec((1, nn, 3), lambda i, j: (i, 0, 0)),
        ],
        out_specs=pl.BlockSpec((1, K, mq), lambda i, j: (i, 0, j)),
        out_shape=jax.ShapeDtypeStruct((b, K, m), jnp.int32),
    )(qT, rN)
    return jnp.transpose(out, (0, 2, 1)).reshape(-1)


def _gather_rows(table, idx, dp):
    ri = idx.shape[0]
    rows_w = ri // NW
    n_chunks = rows_w // CHUNK
    mesh = plsc.VectorSubcoreMesh(core_axis_name="c", subcore_axis_name="s")

    @functools.partial(
        pl.kernel, mesh=mesh,
        out_type=jax.ShapeDtypeStruct((ri, dp), jnp.float32),
        scratch_types=[
            pltpu.VMEM((CHUNK,), jnp.int32),
            pltpu.VMEM((CHUNK, dp), jnp.float32),
            pltpu.SemaphoreType.DMA,
        ],
    )
    def gk(idx_hbm, table_hbm, out_hbm, idx_v, rows_v, sem):
        wid = lax.axis_index("s") * 2 + lax.axis_index("c")
        base = wid * rows_w

        def body(c, carry):
            off = base + c * CHUNK
            pltpu.sync_copy(idx_hbm.at[pl.ds(off, CHUNK)], idx_v)
            pltpu.async_copy(table_hbm.at[idx_v], rows_v, sem).wait()
            pltpu.sync_copy(rows_v, out_hbm.at[pl.ds(off, CHUNK)])
            return carry

        lax.fori_loop(0, n_chunks, body, 0)

    return gk(idx, table)


def _expand_mats(c):
    col = lax.broadcasted_iota(jnp.int32, (c, c * WN), 1)
    row = lax.broadcasted_iota(jnp.int32, (c, c * WN), 0)
    e = (col // WN == row).astype(jnp.float32)
    colt = lax.broadcasted_iota(jnp.int32, (WN, c * WN), 1)
    rowt = lax.broadcasted_iota(jnp.int32, (WN, c * WN), 0)
    t = (colt % WN == rowt).astype(jnp.float32)
    return e, t


def _down_body(rows_ref, nx_ref, wnT_ref, wnb_ref, ltx_ref, ltf_ref, lb_ref,
               o_ref, *, c, dp):
    ex, tx = _expand_mats(3)
    ef, tf = _expand_mats(c)
    nx = nx_ref[...]
    wnT = wnT_ref[...]
    wnb = wnb_ref[...]
    ax = None
    af = None
    for k in range(K):
        base = k * dp
        xk = rows_ref[:, base:base + 3] - nx
        fk = rows_ref[:, base + 3:base + 3 + c]
        wk = jnp.maximum(_dot(xk, wnT) + wnb, 0.0)
        cx = _dot(xk, ex) * _dot(wk, tx)
        cf = _dot(fk, ef) * _dot(wk, tf)
        ax = cx if ax is None else ax + cx
        af = cf if af is None else af + cf
    o_ref[...] = (_dot(ax, ltx_ref[...]) + _dot(af, ltf_ref[...])
                  + lb_ref[...])


def _bn_relu_body(p_ref, g_ref, bb_ref, o_ref):
    pre = p_ref[...]
    mu = jnp.mean(pre, axis=0, keepdims=True)
    var = jnp.mean((pre - mu) ** 2, axis=0, keepdims=True)
    y = (pre - mu) / jnp.sqrt(var + EPS) * g_ref[...] + bb_ref[...]
    o_ref[...] = jnp.maximum(y, 0.0)


def _bn_relu(pre, g, bb):
    r = pre.shape[0]
    return pl.pallas_call(
        _bn_relu_body,
        out_shape=jax.ShapeDtypeStruct((r, OC), jnp.float32),
    )(pre, g, bb)


def _down_dense(rows2d, newx, wnT, wnb, ltx, ltf, lb, g, bb, c, dp, rb):
    r = rows2d.shape[0]
    pre = pl.pallas_call(
        functools.partial(_down_body, c=c, dp=dp),
        grid=(r // rb,),
        in_specs=[
            pl.BlockSpec((rb, K * dp), lambda i: (i, 0)),
            pl.BlockSpec((rb, 3), lambda i: (i, 0)),
            pl.BlockSpec(wnT.shape, lambda i: (0, 0)),
            pl.BlockSpec(wnb.shape, lambda i: (0, 0)),
            pl.BlockSpec(ltx.shape, lambda i: (0, 0)),
            pl.BlockSpec(ltf.shape, lambda i: (0, 0)),
            pl.BlockSpec(lb.shape, lambda i: (0, 0)),
        ],
        out_specs=pl.BlockSpec((rb, OC), lambda i: (i, 0)),
        out_shape=jax.ShapeDtypeStruct((r, OC), jnp.float32),
    )(rows2d, newx, wnT, wnb, ltx, ltf, lb)
    return _bn_relu(pre, g, bb)


def _upi_body(rows_ref, x1_ref, f1_ref, w1f_ref, w1i_ref, w1x_ref, b1_ref,
              o_ref, *, dp):
    wsum = None
    interp = None
    for k in range(K):
        base = k * dp
        dk = None
        for d in range(3):
            dd = x1_ref[:, d:d + 1] - rows_ref[:, base + d:base + d + 1]
            dk = dd * dd if dk is None else dk + dd * dd
        wk = 1.0 / (dk + 1e-8)
        fk = rows_ref[:, base + 3:base + 3 + OC]
        wsum = wk if wsum is None else wsum + wk
        interp = fk * wk if interp is None else interp + fk * wk
    interp = interp / wsum
    pre = (_dot(f1_ref[...], w1f_ref[...])
           + _dot(interp, w1i_ref[...])
           + _dot(x1_ref[...], w1x_ref[...]) + b1_ref[...])
    o_ref[...] = pre


def _up_interp(rows2d, x1, f1, w1f, w1i, w1x, b1, dp, rb):
    r = rows2d.shape[0]
    c1 = f1.shape[1]
    grid = (r // rb,)
    return pl.pallas_call(
        functools.partial(_upi_body, dp=dp),
        grid=grid,
        in_specs=[
            pl.BlockSpec((rb, K * dp), lambda i: (i, 0)),
            pl.BlockSpec((rb, 3), lambda i: (i, 0)),
            pl.BlockSpec((rb, c1), lambda i: (i, 0)),
            pl.BlockSpec(w1f.shape, lambda i: (0, 0)),
            pl.BlockSpec(w1i.shape, lambda i: (0, 0)),
            pl.BlockSpec(w1x.shape, lambda i: (0, 0)),
            pl.BlockSpec(b1.shape, lambda i: (0, 0)),
        ],
        out_specs=pl.BlockSpec((rb, OC), lambda i: (i, 0)),
        out_shape=jax.ShapeDtypeStruct((r, OC), jnp.float32),
    )(rows2d, x1, f1, w1f, w1i, w1x, b1)


def _bn_rows(x, g, bb):
    mu = jnp.mean(x, axis=0, keepdims=True)
    var = jnp.mean((x - mu) ** 2, axis=0, keepdims=True)
    return (x - mu) / jnp.sqrt(var + EPS) * g + bb


def _upm_body(p_ref, c2T_ref, b2_ref, g1_ref, bb1_ref, g2_ref, bb2_ref,
              o_ref):
    h1 = jnp.maximum(_bn_rows(p_ref[...], g1_ref[...], bb1_ref[...]), 0.0)
    p2 = _dot(h1, c2T_ref[...]) + b2_ref[...]
    o_ref[...] = jnp.maximum(_bn_rows(p2, g2_ref[...], bb2_ref[...]), 0.0)


def _up_mlp(pre1, c2T, b2, g1, bb1, g2, bb2):
    r = pre1.shape[0]
    return pl.pallas_call(
        _upm_body,
        out_shape=jax.ShapeDtypeStruct((r, OC), jnp.float32),
    )(pre1, c2T, b2, g1, bb1, g2, bb2)


def _upmh_body(p_ref, c2T_ref, b2_ref, g1_ref, bb1_ref, g2_ref, bb2_ref,
               f1T_ref, f1b_ref, fg_ref, fb_ref, f2T_ref, f2b_ref, o_ref):
    h1 = jnp.maximum(_bn_rows(p_ref[...], g1_ref[...], bb1_ref[...]), 0.0)
    p2 = _dot(h1, c2T_ref[...]) + b2_ref[...]
    h2 = jnp.maximum(_bn_rows(p2, g2_ref[...], bb2_ref[...]), 0.0)
    o32 = _dot(h2, f1T_ref[...]) + f1b_ref[...]
    o32 = _bn_rows(o32, fg_ref[...], fb_ref[...])
    o1 = _dot(o32, f2T_ref[...]) + f2b_ref[...]
    o_ref[...] = jax.nn.sigmoid(o1)


def _up_mlp_head(pre1, c2T, b2, g1, bb1, g2, bb2, f1T, f1b, fg, fb, f2T,
                 f2b):
    r = pre1.shape[0]
    return pl.pallas_call(
        _upmh_body,
        out_shape=jax.ShapeDtypeStruct((r, 1), jnp.float32),
    )(pre1, c2T, b2, g1, bb1, g2, bb2, f1T, f1b, fg, fb, f2T, f2b)


def _down_params(params, name, c):
    cin = c + 3
    wnT = params[name + "_wn_W"].T
    wnb = params[name + "_wn_b"].reshape(1, WN)
    linT = params[name + "_lin_W"].T.reshape(cin, WN, OC)
    ltx = linT[:3].reshape(3 * WN, OC)
    ltf = linT[3:].reshape(c * WN, OC)
    lb = params[name + "_lin_b"].reshape(1, OC)
    g = params[name + "_bn_g"].reshape(1, OC)
    bb = params[name + "_bn_b"].reshape(1, OC)
    return wnT, wnb, ltx, ltf, lb, g, bb


def _up_params(params, name, c1):
    c1T = params[name + "_c1_W"].T
    w1f = c1T[:c1]
    w1i = c1T[c1:c1 + OC]
    w1x = c1T[c1 + OC:]
    b1 = params[name + "_c1_b"].reshape(1, OC)
    c2T = params[name + "_c2_W"].T
    b2 = params[name + "_c2_b"].reshape(1, OC)
    g1 = params[name + "_bn1_g"].reshape(1, OC)
    bb1 = params[name + "_bn1_b"].reshape(1, OC)
    g2 = params[name + "_bn2_g"].reshape(1, OC)
    bb2 = params[name + "_bn2_b"].reshape(1, OC)
    return w1f, w1i, w1x, b1, c2T, b2, g1, bb1, g2, bb2


def _pad_cols(x, dp):
    return jnp.pad(x, ((0, 0), (0, dp - x.shape[1])))


def kernel(xyz, flow, params):
    b, _, n = xyz.shape
    xyzN = jnp.transpose(xyz, (0, 2, 1))
    flowN = jnp.transpose(flow, (0, 2, 1))

    q1T = xyz[:, :, ::4]
    x1N = xyzN[:, ::4, :]
    idx1 = _knn(q1T, xyzN, mq=128)
    t1 = _pad_cols(jnp.concatenate([xyzN, flowN], -1).reshape(b * n, 6), 128)
    rows1 = _gather_rows(t1, idx1, 128)[:, :16].reshape(b * 2048, K * 16)
    wnT, wnb, ltx, ltf, lb, g, bbp = _down_params(params, "l1", 3)
    feat1 = _down_dense(rows1, x1N.reshape(b * 2048, 3),
                        wnT, wnb, ltx, ltf, lb, g, bbp, 3, 16, 1024)

    q2T = q1T[:, :, ::4]
    x2N = x1N[:, ::4, :]
    idx2 = _knn(q2T, x1N, mq=512)
    t2 = _pad_cols(jnp.concatenate([x1N.reshape(b * 2048, 3), feat1], -1), 128)
    rows2 = _gather_rows(t2, idx2, 128).reshape(b * 512, K * 128)
    wnT, wnb, ltx, ltf, lb, g, bbp = _down_params(params, "l2", OC)
    feat2 = _down_dense(rows2, x2N.reshape(b * 512, 3),
                        wnT, wnb, ltx, ltf, lb, g, bbp, OC, 128, 512)

    q3T = q2T[:, :, ::2]
    x3N = x2N[:, ::2, :]
    idx3 = _knn(q3T, x2N, mq=256)
    t3 = _pad_cols(jnp.concatenate([x2N.reshape(b * 512, 3), feat2], -1), 128)
    rows3 = _gather_rows(t3, idx3, 128).reshape(b * 256, K * 128)
    wnT, wnb, ltx, ltf, lb, g, bbp = _down_params(params, "l3", OC)
    feat3 = _down_dense(rows3, x3N.reshape(b * 256, 3),
                        wnT, wnb, ltx, ltf, lb, g, bbp, OC, 128, 512)

    idxu3 = _knn(q2T, x3N, mq=512)
    tu3 = _pad_cols(jnp.concatenate([x3N.reshape(b * 256, 3), feat3], -1), 128)
    rowsu3 = _gather_rows(tu3, idxu3, 128).reshape(b * 512, K * 128)
    w1f, w1i, w1x, b1, c2T, b2, g1, bb1, g2, bb2 = _up_params(params, "u3", OC)
    pre1 = _up_interp(rowsu3, x2N.reshape(b * 512, 3), feat2,
                      w1f, w1i, w1x, b1, 128, 1024)
    up3 = _up_mlp(pre1, c2T, b2, g1, bb1, g2, bb2)

    idxu2 = _knn(q1T, x2N, mq=512)
    tu2 = _pad_cols(jnp.concatenate([x2N.reshape(b * 512, 3), up3], -1), 128)
    rowsu2 = _gather_rows(tu2, idxu2, 128).reshape(b * 2048, K * 128)
    w1f, w1i, w1x, b1, c2T, b2, g1, bb1, g2, bb2 = _up_params(params, "u2", OC)
    pre1 = _up_interp(rowsu2, x1N.reshape(b * 2048, 3), feat1,
                      w1f, w1i, w1x, b1, 128, 1024)
    up2 = _up_mlp(pre1, c2T, b2, g1, bb1, g2, bb2)

    idxu1 = _knn(xyz, x1N, mq=512)
    tu1 = _pad_cols(jnp.concatenate([x1N.reshape(b * 2048, 3), up2], -1), 128)
    rowsu1 = _gather_rows(tu1, idxu1, 128).reshape(b * n, K * 128)
    w1f, w1i, w1x, b1, c2T, b2, g1, bb1, g2, bb2 = _up_params(params, "u1", 3)
    pre1 = _up_interp(rowsu1, xyzN.reshape(b * n, 3),
                      xyzN.reshape(b * n, 3),
                      w1f, w1i, w1x, b1, 128, 1024)
    out = _up_mlp_head(pre1, c2T, b2, g1, bb1, g2, bb2,
                       params["f1_W"].T, params["f1_b"].reshape(1, 32),
                       params["fbn_g"].reshape(1, 32),
                       params["fbn_b"].reshape(1, 32),
                       params["f2_W"].T, params["f2_b"].reshape(1, 1))
    return out.reshape(b, n)


# speedup vs baseline: 13.5632x; 1.2732x over previous
import functools

import jax
import jax.numpy as jnp
from jax import lax
from jax.experimental import pallas as pl
from jax.experimental.pallas import tpu as pltpu
from jax.experimental.pallas import tpu_sc as plsc

EPS = 1e-5
K = 16
WN = 16
OC = 64
NW = 32
CHUNK = 128

def _dot(a, b):
    return jax.lax.dot_general(
        a.astype(jnp.bfloat16), b.astype(jnp.bfloat16),
        (((1,), (0,)), ((), ())), preferred_element_type=jnp.float32)


def _knn_body(qT_ref, rN_ref, o_ref, *, nn):
    b = pl.program_id(0)
    mq = qT_ref.shape[2]
    inf = jnp.float32(float("inf"))
    qs = None
    rs = None
    qr = None
    for d in range(3):
        qd = qT_ref[0, d:d + 1, :]
        rd = rN_ref[0, :, d:d + 1]
        qdb = qd.astype(jnp.bfloat16).astype(jnp.float32)
        rdb = rd.astype(jnp.bfloat16).astype(jnp.float32)
        qs = qd * qd if qs is None else qs + qd * qd
        rs = rd * rd if rs is None else rs + rd * rd
        qr = rdb * qdb if qr is None else qr + rdb * qdb
    d2 = (qs - 2.0 * qr) + rs
    sub_iota = lax.broadcasted_iota(jnp.int32, (nn, mq), 0)
    off = b * nn
    for j in range(K):
        idxj = jnp.argmin(d2, axis=0).astype(jnp.int32).reshape(1, mq)
        o_ref[0, j:j + 1, :] = idxj + off
        if j < K - 1:
            d2 = jnp.where(sub_iota == idxj, inf, d2)


def _knn(qT, rN, mq):
    b, _, m = qT.shape
    nn = rN.shape[1]
    out = pl.pallas_call(
        functools.partial(_knn_body, nn=nn),
        grid=(b, m // mq),
        in_specs=[
            pl.BlockSpec((1, 3, mq), lambda i, j: (i, 0, j)),
            pl.BlockSpec((1, nn, 3), lambda i, j: (i, 0, 0)),
        ],
        out_specs=pl.BlockSpec((1, K, mq), lambda i, j: (i, 0, j)),
        out_shape=jax.ShapeDtypeStruct((b, K, m), jnp.int32),
    )(qT, rN)
    return jnp.transpose(out, (0, 2, 1)).reshape(-1)


def _gather_rows(table, idx, dp):
    ri = idx.shape[0]
    rows_w = ri // NW
    n_chunks = rows_w // CHUNK
    grp = 4 if n_chunks % 4 == 0 else 2
    idx2d = idx.reshape(ri // CHUNK, CHUNK)
    mesh = plsc.VectorSubcoreMesh(core_axis_name="c", subcore_axis_name="s")

    @functools.partial(
        pl.kernel, mesh=mesh,
        out_type=jax.ShapeDtypeStruct((ri, dp), jnp.float32),
        scratch_types=(
            [pltpu.VMEM((n_chunks, CHUNK), jnp.int32)]
            + [pltpu.VMEM((CHUNK, dp), jnp.float32)] * grp
            + [pltpu.SemaphoreType.DMA] * grp
        ),
    )
    def gk(idx_hbm, table_hbm, out_hbm, idx_all, *rest):
        rows_v = rest[:grp]
        sems = rest[grp:]
        wid = lax.axis_index("s") * 2 + lax.axis_index("c")
        base = wid * rows_w
        pltpu.sync_copy(idx_hbm.at[pl.ds(wid * n_chunks, n_chunks)],
                        idx_all)

        def body(g, carry):
            c = g * grp
            handles = []
            for b_ in range(grp):
                h = pltpu.async_copy(
                    table_hbm.at[idx_all.at[c + b_]], rows_v[b_], sems[b_])
                handles.append(h)
            for b_ in range(grp):
                handles[b_].wait()
                off = base + (c + b_) * CHUNK
                pltpu.sync_copy(rows_v[b_], out_hbm.at[pl.ds(off, CHUNK)])
            return carry

        lax.fori_loop(0, n_chunks // grp, body, 0)

    return gk(idx2d, table)


def _expand_mats(c):
    col = lax.broadcasted_iota(jnp.int32, (c, c * WN), 1)
    row = lax.broadcasted_iota(jnp.int32, (c, c * WN), 0)
    e = (col // WN == row).astype(jnp.float32)
    colt = lax.broadcasted_iota(jnp.int32, (WN, c * WN), 1)
    rowt = lax.broadcasted_iota(jnp.int32, (WN, c * WN), 0)
    t = (colt % WN == rowt).astype(jnp.float32)
    return e, t


def _down_body(rows_ref, nx_ref, wnT_ref, wnb_ref, ltx_ref, ltf_ref, lb_ref,
               o_ref, *, c, dp):
    ex, tx = _expand_mats(3)
    ef, tf = _expand_mats(c)
    nx = nx_ref[...]
    wnT = wnT_ref[...]
    wnb = wnb_ref[...]
    ax = None
    af = None
    for k in range(K):
        base = k * dp
        xk = rows_ref[:, base:base + 3] - nx
        fk = rows_ref[:, base + 3:base + 3 + c]
        wk = jnp.maximum(_dot(xk, wnT) + wnb, 0.0)
        cx = _dot(xk, ex) * _dot(wk, tx)
        cf = _dot(fk, ef) * _dot(wk, tf)
        ax = cx if ax is None else ax + cx
        af = cf if af is None else af + cf
    o_ref[...] = (_dot(ax, ltx_ref[...]) + _dot(af, ltf_ref[...])
                  + lb_ref[...])


def _bn_relu_body(p_ref, g_ref, bb_ref, o_ref):
    pre = p_ref[...]
    mu = jnp.mean(pre, axis=0, keepdims=True)
    var = jnp.mean((pre - mu) ** 2, axis=0, keepdims=True)
    y = (pre - mu) / jnp.sqrt(var + EPS) * g_ref[...] + bb_ref[...]
    o_ref[...] = jnp.maximum(y, 0.0)


def _bn_relu(pre, g, bb):
    r = pre.shape[0]
    return pl.pallas_call(
        _bn_relu_body,
        out_shape=jax.ShapeDtypeStruct((r, OC), jnp.float32),
    )(pre, g, bb)


def _down_dense(rows2d, newx, wnT, wnb, ltx, ltf, lb, g, bb, c, dp, rb):
    r = rows2d.shape[0]
    pre = pl.pallas_call(
        functools.partial(_down_body, c=c, dp=dp),
        grid=(r // rb,),
        in_specs=[
            pl.BlockSpec((rb, K * dp), lambda i: (i, 0)),
            pl.BlockSpec((rb, 3), lambda i: (i, 0)),
            pl.BlockSpec(wnT.shape, lambda i: (0, 0)),
            pl.BlockSpec(wnb.shape, lambda i: (0, 0)),
            pl.BlockSpec(ltx.shape, lambda i: (0, 0)),
            pl.BlockSpec(ltf.shape, lambda i: (0, 0)),
            pl.BlockSpec(lb.shape, lambda i: (0, 0)),
        ],
        out_specs=pl.BlockSpec((rb, OC), lambda i: (i, 0)),
        out_shape=jax.ShapeDtypeStruct((r, OC), jnp.float32),
    )(rows2d, newx, wnT, wnb, ltx, ltf, lb)
    return _bn_relu(pre, g, bb)


def _upi_body(rows_ref, x1_ref, f1_ref, w1f_ref, w1i_ref, w1x_ref, b1_ref,
              o_ref, *, dp):
    wsum = None
    interp = None
    for k in range(K):
        base = k * dp
        dk = None
        for d in range(3):
            dd = x1_ref[:, d:d + 1] - rows_ref[:, base + d:base + d + 1]
            dk = dd * dd if dk is None else dk + dd * dd
        wk = 1.0 / (dk + 1e-8)
        fk = rows_ref[:, base + 3:base + 3 + OC]
        wsum = wk if wsum is None else wsum + wk
        interp = fk * wk if interp is None else interp + fk * wk
    interp = interp / wsum
    pre = (_dot(f1_ref[...], w1f_ref[...])
           + _dot(interp, w1i_ref[...])
           + _dot(x1_ref[...], w1x_ref[...]) + b1_ref[...])
    o_ref[...] = pre


def _up_interp(rows2d, x1, f1, w1f, w1i, w1x, b1, dp, rb):
    r = rows2d.shape[0]
    c1 = f1.shape[1]
    grid = (r // rb,)
    return pl.pallas_call(
        functools.partial(_upi_body, dp=dp),
        grid=grid,
        in_specs=[
            pl.BlockSpec((rb, K * dp), lambda i: (i, 0)),
            pl.BlockSpec((rb, 3), lambda i: (i, 0)),
            pl.BlockSpec((rb, c1), lambda i: (i, 0)),
            pl.BlockSpec(w1f.shape, lambda i: (0, 0)),
            pl.BlockSpec(w1i.shape, lambda i: (0, 0)),
            pl.BlockSpec(w1x.shape, lambda i: (0, 0)),
            pl.BlockSpec(b1.shape, lambda i: (0, 0)),
        ],
        out_specs=pl.BlockSpec((rb, OC), lambda i: (i, 0)),
        out_shape=jax.ShapeDtypeStruct((r, OC), jnp.float32),
    )(rows2d, x1, f1, w1f, w1i, w1x, b1)


def _bn_rows(x, g, bb):
    mu = jnp.mean(x, axis=0, keepdims=True)
    var = jnp.mean((x - mu) ** 2, axis=0, keepdims=True)
    return (x - mu) / jnp.sqrt(var + EPS) * g + bb


def _upm_body(p_ref, c2T_ref, b2_ref, g1_ref, bb1_ref, g2_ref, bb2_ref,
              o_ref):
    h1 = jnp.maximum(_bn_rows(p_ref[...], g1_ref[...], bb1_ref[...]), 0.0)
    p2 = _dot(h1, c2T_ref[...]) + b2_ref[...]
    o_ref[...] = jnp.maximum(_bn_rows(p2, g2_ref[...], bb2_ref[...]), 0.0)


def _up_mlp(pre1, c2T, b2, g1, bb1, g2, bb2):
    r = pre1.shape[0]
    return pl.pallas_call(
        _upm_body,
        out_shape=jax.ShapeDtypeStruct((r, OC), jnp.float32),
    )(pre1, c2T, b2, g1, bb1, g2, bb2)


def _upmh_body(p_ref, c2T_ref, b2_ref, g1_ref, bb1_ref, g2_ref, bb2_ref,
               f1T_ref, f1b_ref, fg_ref, fb_ref, f2T_ref, f2b_ref, o_ref):
    h1 = jnp.maximum(_bn_rows(p_ref[...], g1_ref[...], bb1_ref[...]), 0.0)
    p2 = _dot(h1, c2T_ref[...]) + b2_ref[...]
    h2 = jnp.maximum(_bn_rows(p2, g2_ref[...], bb2_ref[...]), 0.0)
    o32 = _dot(h2, f1T_ref[...]) + f1b_ref[...]
    o32 = _bn_rows(o32, fg_ref[...], fb_ref[...])
    o1 = _dot(o32, f2T_ref[...]) + f2b_ref[...]
    o_ref[...] = jax.nn.sigmoid(o1)


def _up_mlp_head(pre1, c2T, b2, g1, bb1, g2, bb2, f1T, f1b, fg, fb, f2T,
                 f2b):
    r = pre1.shape[0]
    return pl.pallas_call(
        _upmh_body,
        out_shape=jax.ShapeDtypeStruct((r, 1), jnp.float32),
    )(pre1, c2T, b2, g1, bb1, g2, bb2, f1T, f1b, fg, fb, f2T, f2b)


def _down_params(params, name, c):
    cin = c + 3
    wnT = params[name + "_wn_W"].T
    wnb = params[name + "_wn_b"].reshape(1, WN)
    linT = params[name + "_lin_W"].T.reshape(cin, WN, OC)
    ltx = linT[:3].reshape(3 * WN, OC)
    ltf = linT[3:].reshape(c * WN, OC)
    lb = params[name + "_lin_b"].reshape(1, OC)
    g = params[name + "_bn_g"].reshape(1, OC)
    bb = params[name + "_bn_b"].reshape(1, OC)
    return wnT, wnb, ltx, ltf, lb, g, bb


def _up_params(params, name, c1):
    c1T = params[name + "_c1_W"].T
    w1f = c1T[:c1]
    w1i = c1T[c1:c1 + OC]
    w1x = c1T[c1 + OC:]
    b1 = params[name + "_c1_b"].reshape(1, OC)
    c2T = params[name + "_c2_W"].T
    b2 = params[name + "_c2_b"].reshape(1, OC)
    g1 = params[name + "_bn1_g"].reshape(1, OC)
    bb1 = params[name + "_bn1_b"].reshape(1, OC)
    g2 = params[name + "_bn2_g"].reshape(1, OC)
    bb2 = params[name + "_bn2_b"].reshape(1, OC)
    return w1f, w1i, w1x, b1, c2T, b2, g1, bb1, g2, bb2


def _pad_cols(x, dp):
    return jnp.pad(x, ((0, 0), (0, dp - x.shape[1])))


def kernel(xyz, flow, params):
    b, _, n = xyz.shape
    xyzN = jnp.transpose(xyz, (0, 2, 1))
    flowN = jnp.transpose(flow, (0, 2, 1))

    q1T = xyz[:, :, ::4]
    x1N = xyzN[:, ::4, :]
    idx1 = _knn(q1T, xyzN, mq=128)
    t1 = _pad_cols(jnp.concatenate([xyzN, flowN], -1).reshape(b * n, 6), 128)
    rows1 = _gather_rows(t1, idx1, 128)[:, :16].reshape(b * 2048, K * 16)
    wnT, wnb, ltx, ltf, lb, g, bbp = _down_params(params, "l1", 3)
    feat1 = _down_dense(rows1, x1N.reshape(b * 2048, 3),
                        wnT, wnb, ltx, ltf, lb, g, bbp, 3, 16, 1024)

    q2T = q1T[:, :, ::4]
    x2N = x1N[:, ::4, :]
    idx2 = _knn(q2T, x1N, mq=512)
    t2 = _pad_cols(jnp.concatenate([x1N.reshape(b * 2048, 3), feat1], -1), 128)
    rows2 = _gather_rows(t2, idx2, 128).reshape(b * 512, K * 128)
    wnT, wnb, ltx, ltf, lb, g, bbp = _down_params(params, "l2", OC)
    feat2 = _down_dense(rows2, x2N.reshape(b * 512, 3),
                        wnT, wnb, ltx, ltf, lb, g, bbp, OC, 128, 512)

    q3T = q2T[:, :, ::2]
    x3N = x2N[:, ::2, :]
    idx3 = _knn(q3T, x2N, mq=256)
    t3 = _pad_cols(jnp.concatenate([x2N.reshape(b * 512, 3), feat2], -1), 128)
    rows3 = _gather_rows(t3, idx3, 128).reshape(b * 256, K * 128)
    wnT, wnb, ltx, ltf, lb, g, bbp = _down_params(params, "l3", OC)
    feat3 = _down_dense(rows3, x3N.reshape(b * 256, 3),
                        wnT, wnb, ltx, ltf, lb, g, bbp, OC, 128, 512)

    idxu3 = _knn(q2T, x3N, mq=512)
    tu3 = _pad_cols(jnp.concatenate([x3N.reshape(b * 256, 3), feat3], -1), 128)
    rowsu3 = _gather_rows(tu3, idxu3, 128).reshape(b * 512, K * 128)
    w1f, w1i, w1x, b1, c2T, b2, g1, bb1, g2, bb2 = _up_params(params, "u3", OC)
    pre1 = _up_interp(rowsu3, x2N.reshape(b * 512, 3), feat2,
                      w1f, w1i, w1x, b1, 128, 1024)
    up3 = _up_mlp(pre1, c2T, b2, g1, bb1, g2, bb2)

    idxu2 = _knn(q1T, x2N, mq=512)
    tu2 = _pad_cols(jnp.concatenate([x2N.reshape(b * 512, 3), up3], -1), 128)
    rowsu2 = _gather_rows(tu2, idxu2, 128).reshape(b * 2048, K * 128)
    w1f, w1i, w1x, b1, c2T, b2, g1, bb1, g2, bb2 = _up_params(params, "u2", OC)
    pre1 = _up_interp(rowsu2, x1N.reshape(b * 2048, 3), feat1,
                      w1f, w1i, w1x, b1, 128, 1024)
    up2 = _up_mlp(pre1, c2T, b2, g1, bb1, g2, bb2)

    idxu1 = _knn(xyz, x1N, mq=512)
    tu1 = _pad_cols(jnp.concatenate([x1N.reshape(b * 2048, 3), up2], -1), 128)
    rowsu1 = _gather_rows(tu1, idxu1, 128).reshape(b * n, K * 128)
    w1f, w1i, w1x, b1, c2T, b2, g1, bb1, g2, bb2 = _up_params(params, "u1", 3)
    pre1 = _up_interp(rowsu1, xyzN.reshape(b * n, 3),
                      xyzN.reshape(b * n, 3),
                      w1f, w1i, w1x, b1, 128, 1024)
    out = _up_mlp_head(pre1, c2T, b2, g1, bb1, g2, bb2,
                       params["f1_W"].T, params["f1_b"].reshape(1, 32),
                       params["fbn_g"].reshape(1, 32),
                       params["fbn_b"].reshape(1, 32),
                       params["f2_W"].T, params["f2_b"].reshape(1, 1))
    return out.reshape(b, n)
